# baseline (speedup 1.0000x reference)
import functools

import jax
import jax.numpy as jnp
from jax import lax
from jax.experimental import pallas as pl
from jax.experimental.pallas import tpu as pltpu
from jax.experimental.pallas import tpu_sc as plsc

N = 10000
D = 128
T = 200
BATCH = 1024
NC, NS, L = 2, 16, 16
EH = 160000
K = 96
NCHUNK = 112
SEGC = 16
NSEG = NCHUNK // SEGC
EPT = NCHUNK * K
EP = EPT * NS
PADN = EP - EH
SROWS = 10240
CROWS = N + 16
CFLAT = CROWS * T
CPT = CFLAT // NS
NT = N * T
CO = NT // NS
BPT = BATCH // (NC * NS)

_mesh = plsc.VectorSubcoreMesh(
    core_axis_name="c", subcore_axis_name="s", num_cores=NC, num_subcores=NS)

_f32 = jnp.float32


NDEG = SROWS


@functools.partial(
    pl.kernel,
    out_type=jax.ShapeDtypeStruct((NC * NDEG,), _f32),
    mesh=_mesh,
    scratch_types=[
        pltpu.VMEM((NCHUNK, K), jnp.int32),
        pltpu.VMEM((K,), _f32),
        pltpu.VMEM((NDEG // NS,), _f32),
        pltpu.VMEM_SHARED((NDEG,), _f32),
    ],
)
def _sc_degree(row_hbm, out_hbm, row_v, ones_v, zb_v, acc_sh):
    c = lax.axis_index("c")
    s = lax.axis_index("s")
    pltpu.sync_copy(row_hbm.at[c, s], row_v)
    zeros = jnp.zeros((L,), _f32)
    ones = jnp.ones((L,), _f32)
    def zb(i, carry):
        zb_v[pl.ds(i * L, L)] = zeros
        return carry
    lax.fori_loop(0, NDEG // NS // L, zb, 0)
    def ob(i, carry):
        ones_v[pl.ds(i * L, L)] = ones
        return carry
    lax.fori_loop(0, K // L, ob, 0)
    pltpu.sync_copy(zb_v, acc_sh.at[pl.ds(s * (NDEG // NS), NDEG // NS)])
    plsc.subcore_barrier()
    def body(j, carry):
        pltpu.sync_copy(ones_v, acc_sh.at[row_v.at[j]], add=True)
        return carry
    lax.fori_loop(0, NCHUNK, body, 0)
    plsc.subcore_barrier()
    pltpu.sync_copy(acc_sh.at[pl.ds(s * (NDEG // NS), NDEG // NS)], zb_v)
    pltpu.sync_copy(zb_v,
                    out_hbm.at[pl.ds(c * NDEG + s * (NDEG // NS), NDEG // NS)])


NPASS = 2
PROWS = N // NPASS
CDUMP = PROWS * T
CFH = CDUMP + 1600
CTILE = CFH // NS
CCH = 2504
SCAP = 15


@functools.partial(
    pl.kernel,
    out_type=jax.ShapeDtypeStruct((NC * NPASS * CFH,), _f32),
    mesh=_mesh,
    scratch_types=[
        pltpu.VMEM((SEGC, K), jnp.int32),
        pltpu.VMEM((SEGC, K), jnp.int32),
        pltpu.VMEM((SEGC, K), jnp.int32),
        pltpu.VMEM((SEGC, K), _f32),
        pltpu.VMEM((2512,), _f32),
        pltpu.VMEM((2512,), _f32),
        pltpu.VMEM_SHARED((CFH,), _f32),
        pltpu.SemaphoreType.DMA,
        pltpu.SemaphoreType.DMA,
    ],
)
def _sc_buildc(fidx_hbm, col_hbm, dinv_hbm, out_hbm,
               col_sv, fidx_sv, idxp_sv, val_sv, zb_v, cb_v, acc_sh,
               sem, sem2):
    c = lax.axis_index("c")
    s = lax.axis_index("s")
    zeros = jnp.zeros((L,), _f32)
    def zf(i, carry):
        zb_v[pl.ds(i * L, L)] = zeros
        return carry
    lax.fori_loop(0, 2512 // L, zf, 0)
    dinv = dinv_hbm.at[pl.ds(c * N, N)]
    G = K // L
    for p in range(NPASS):
        def zc(i, carry):
            pltpu.sync_copy(zb_v.at[pl.ds(0, CCH)],
                            acc_sh.at[pl.ds(s * CTILE + i * CCH, CCH)])
            return carry
        lax.fori_loop(0, CTILE // CCH, zc, 0)
        plsc.subcore_barrier()
        lo = p * CDUMP
        def seg(g, carry):
            pltpu.sync_copy(col_hbm.at[c, s, pl.ds(g * SEGC, SEGC)], col_sv)
            pltpu.sync_copy(fidx_hbm.at[c, s, pl.ds(g * SEGC, SEGC)], fidx_sv)
            def gat(j, carry2):
                pltpu.async_copy(dinv.at[col_sv.at[j]], val_sv.at[j], sem)
                return carry2
            lax.fori_loop(0, SEGC, gat, 0)
            def clamp(i, carry2):
                j = i // G
                o = (i % G) * L
                fi = fidx_sv[j, pl.ds(o, L)]
                inr = (fi >= lo) & (fi < lo + CDUMP)
                idxp_sv[j, pl.ds(o, L)] = jnp.where(inr, fi - lo, CDUMP)
                return carry2
            lax.fori_loop(0, SEGC * G, clamp, 0)
            def gatw(j, carry2):
                pltpu.make_async_copy(dinv.at[col_sv.at[j]], val_sv.at[j],
                                      sem).wait()
                return carry2
            lax.fori_loop(0, SEGC, gatw, 0)
            def scat(j, carry2):
                pltpu.async_copy(val_sv.at[j], acc_sh.at[idxp_sv.at[j]], sem2,
                                 add=True)
                return carry2
            lax.fori_loop(0, SEGC, scat, 0)
            def scatw(j, carry2):
                pltpu.make_async_copy(val_sv.at[j], acc_sh.at[idxp_sv.at[j]],
                                      sem2).wait()
                return carry2
            lax.fori_loop(0, SEGC, scatw, 0)
            return carry
        lax.fori_loop(0, NSEG, seg, 0)
        plsc.subcore_barrier()
        obase = c * (NPASS * CFH) + p * CFH + s * CTILE
        def co(i, carry):
            bounce = cb_v.at[pl.ds(0, CCH)]
            pltpu.sync_copy(acc_sh.at[pl.ds(s * CTILE + i * CCH, CCH)], bounce)
            pltpu.sync_copy(bounce, out_hbm.at[pl.ds(obase + i * CCH, CCH)])
            return carry
        lax.fori_loop(0, CTILE // CCH, co, 0)
        plsc.subcore_barrier()


@functools.partial(
    pl.kernel,
    out_type=jax.ShapeDtypeStruct((NC, SROWS, D), _f32),
    mesh=_mesh,
    scratch_types=[
        pltpu.VMEM((SEGC, K), jnp.int32),
        pltpu.VMEM((SEGC, K), jnp.int32),
        pltpu.VMEM((2, K, D), _f32),
        pltpu.VMEM_SHARED((SROWS, D), _f32),
        pltpu.SemaphoreType.DMA,
        pltpu.SemaphoreType.DMA,
    ],
)
def _sc_spmm(zeros_hbm, xs_hbm, col_hbm, row_hbm, out_hbm,
             col_sv, row_sv, gb_v, acc_sh, sem0, sem1):
    c = lax.axis_index("c")
    s = lax.axis_index("s")
    RPT = SROWS // NS
    pltpu.sync_copy(zeros_hbm, acc_sh.at[pl.ds(s * RPT, RPT)])
    plsc.subcore_barrier()
    xs = xs_hbm.at[c]
    def seg(g, carry):
        pltpu.sync_copy(col_hbm.at[c, s, pl.ds(g * SEGC, SEGC)], col_sv)
        pltpu.sync_copy(row_hbm.at[c, s, pl.ds(g * SEGC, SEGC)], row_sv)
        pltpu.async_copy(xs.at[col_sv.at[0]], gb_v.at[0], sem0)
        def body(j, carry2):
            b = j % 2
            nxt = j + 1
            @pl.when(jnp.logical_and(nxt < SEGC, b == 0))
            def _():
                pltpu.async_copy(xs.at[col_sv.at[nxt]], gb_v.at[1], sem1)
            @pl.when(jnp.logical_and(nxt < SEGC, b == 1))
            def _():
                pltpu.async_copy(xs.at[col_sv.at[nxt]], gb_v.at[0], sem0)
            @pl.when(b == 0)
            def _():
                pltpu.make_async_copy(xs.at[col_sv.at[j]], gb_v.at[0],
                                      sem0).wait()
            @pl.when(b == 1)
            def _():
                pltpu.make_async_copy(xs.at[col_sv.at[j]], gb_v.at[1],
                                      sem1).wait()
            pltpu.sync_copy(gb_v.at[b], acc_sh.at[row_sv.at[j]], add=True)
            return carry2
        lax.fori_loop(0, SEGC, body, 0)
        return carry
    lax.fori_loop(0, NSEG, seg, 0)
    plsc.subcore_barrier()
    pltpu.sync_copy(acc_sh.at[pl.ds(s * RPT, RPT)],
                    out_hbm.at[c, pl.ds(s * RPT, RPT)])


@functools.partial(
    pl.kernel,
    out_type=(jax.ShapeDtypeStruct((BATCH, D), _f32),
              jax.ShapeDtypeStruct((BATCH, D), _f32)),
    mesh=_mesh,
    scratch_types=[
        pltpu.VMEM((BPT,), jnp.int32),
        pltpu.VMEM((BPT, D), _f32),
        pltpu.SemaphoreType.DMA,
    ],
)
def _sc_gather(x_hbm, r_hbm, sub_hbm, rel_hbm, sube_hbm, rele_hbm,
               idx_v, buf_v, sem):
    c = lax.axis_index("c")
    s = lax.axis_index("s")
    base = (s * NC + c) * BPT
    pltpu.sync_copy(sub_hbm.at[pl.ds(base, BPT)], idx_v)
    pltpu.async_copy(x_hbm.at[idx_v], buf_v, sem).wait()
    pltpu.sync_copy(buf_v, sube_hbm.at[pl.ds(base, BPT)])
    pltpu.sync_copy(rel_hbm.at[pl.ds(base, BPT)], idx_v)
    pltpu.async_copy(r_hbm.at[idx_v], buf_v, sem).wait()
    pltpu.sync_copy(buf_v, rele_hbm.at[pl.ds(base, BPT)])


def _tc_pre_body(degp_ref, x_ref, dinv_ref, xs_ref):
    deg = degp_ref[:, :N]
    dinv = jnp.where(deg > 0, lax.rsqrt(jnp.maximum(deg, 1e-12)), 0.0)
    dinv_ref[...] = dinv
    xs_ref[...] = dinv[:, :, None] * x_ref[...][None, :, :]


def _tc_pre(degp, x):
    return pl.pallas_call(
        _tc_pre_body,
        out_shape=(jax.ShapeDtypeStruct((NC, N), _f32),
                   jax.ShapeDtypeStruct((NC, N, D), _f32)),
    )(degp, x)


def _tc_scale_body(dinv_ref, x_ref, xs_ref):
    xs_ref[...] = dinv_ref[...][:, :, None] * x_ref[...][None, :, :]


def _tc_scale(dinv, x):
    return pl.pallas_call(
        _tc_scale_body,
        out_shape=jax.ShapeDtypeStruct((NC, N, D), _f32),
    )(dinv, x)


def _tc_layer_body(want_xsn, x_ref, S_ref, C_ref, dinv_ref, relf_ref,
                   wl_ref, wi_ref, wo_ref, wr_ref, b_ref, g_ref, be_ref,
                   *out_refs):
    x = x_ref[...]
    relf = relf_ref[...]
    rel200 = relf[:T, :]
    dinv = dinv_ref[...]
    res = jnp.zeros((N, D), _f32)
    for h, w_ref in ((0, wi_ref), (1, wo_ref)):
        Rh = jnp.dot(C_ref[h], rel200, preferred_element_type=_f32)
        agg = dinv[h][:, None] * (S_ref[h] - Rh)
        res = res + jnp.dot(agg, w_ref[...], preferred_element_type=_f32)
    loop_res = jnp.dot(x - relf[T, :][None, :], wl_ref[...],
                       preferred_element_type=_f32)
    out = (res + loop_res) * (1.0 / 3.0) + b_ref[...][None, :]
    mean = jnp.mean(out, axis=0)
    var = jnp.mean((out - mean[None, :]) ** 2, axis=0)
    out = (out - mean[None, :]) * lax.rsqrt(var + 1e-5)[None, :] * \
        g_ref[...][None, :] + be_ref[...][None, :]
    out = jnp.tanh(out)
    out_refs[0][...] = out
    out_refs[1][...] = jnp.dot(relf, wr_ref[...],
                               preferred_element_type=_f32)[:T, :]
    if want_xsn:
        out_refs[2][...] = dinv[:, :, None] * out[None, :, :]


def _tc_layer(x, S, C, dinv, relf, wl, wi, wo, wr, b, g, be, want_xsn):
    outs = [jax.ShapeDtypeStruct((N, D), _f32),
            jax.ShapeDtypeStruct((T, D), _f32)]
    if want_xsn:
        outs.append(jax.ShapeDtypeStruct((NC, N, D), _f32))
    return pl.pallas_call(
        functools.partial(_tc_layer_body, want_xsn),
        out_shape=tuple(outs),
    )(x, S, C, dinv, relf, wl, wi, wo, wr, b, g, be)


def kernel(sub, rel, edge_index, edge_type, init_embed, init_rel,
           w_loop1, w_in1, w_out1, w_rel1, loop_rel1, bias1, gamma1, beta1,
           w_loop2, w_in2, w_out2, w_rel2, loop_rel2, bias2, gamma2, beta2):
    ei = edge_index.astype(jnp.int32)
    ety = edge_type.astype(jnp.int32)
    pad_row = jnp.full((PADN,), N, jnp.int32)
    pad_zero = jnp.zeros((PADN,), jnp.int32)
    row_p4 = jnp.stack([jnp.concatenate([ei[0, :EH], pad_row]),
                        jnp.concatenate([ei[0, EH:], pad_row])]
                       ).reshape(NC, NS, NCHUNK, K)
    col_p4 = jnp.stack([jnp.concatenate([ei[1, :EH], pad_zero]),
                        jnp.concatenate([ei[1, EH:], pad_zero])]
                       ).reshape(NC, NS, NCHUNK, K)
    et_p4 = jnp.stack([jnp.concatenate([ety[:EH], pad_zero]),
                       jnp.concatenate([ety[EH:], pad_zero])]
                      ).reshape(NC, NS, NCHUNK, K)
    fidx_p4 = row_p4 * T + et_p4

    degp = _sc_degree(row_p4).reshape(NC, NDEG)
    dinv, xs1 = _tc_pre(degp, init_embed)
    Craw = _sc_buildc(fidx_p4, col_p4, dinv.reshape(-1))
    C = Craw.reshape(NC, NPASS, CFH)[:, :, :CDUMP].reshape(NC, N, T)

    zrows = jnp.zeros((SROWS // NS, D), _f32)
    S1 = _sc_spmm(zrows, xs1, col_p4, row_p4)[:, :N, :]
    relf1 = jnp.concatenate([init_rel, loop_rel1], axis=0)
    x1, r1 = _tc_layer(init_embed, S1, C, dinv, relf1,
                       w_loop1, w_in1, w_out1, w_rel1,
                       bias1, gamma1, beta1, False)
    xs2 = _tc_scale(dinv, x1)

    S2 = _sc_spmm(zrows, xs2, col_p4, row_p4)[:, :N, :]
    relf2 = jnp.concatenate([r1, loop_rel2], axis=0)
    x2, r2 = _tc_layer(x1, S2, C, dinv, relf2,
                       w_loop2, w_in2, w_out2, w_rel2,
                       bias2, gamma2, beta2, False)

    sub_emb, rel_emb = _sc_gather(x2, r2, sub.astype(jnp.int32),
                                  rel.astype(jnp.int32))
    return sub_emb, rel_emb, x2


# speedup vs baseline: 1.6772x; 1.6772x over previous
import functools

import jax
import jax.numpy as jnp
from jax import lax
from jax.experimental import pallas as pl
from jax.experimental.pallas import tpu as pltpu
from jax.experimental.pallas import tpu_sc as plsc

N = 10000
D = 128
T = 200
BATCH = 1024
NC, NS, L = 2, 16, 16
EH = 160000
K = 128
NCHUNK = 80
SEGC = 16
NSEG = NCHUNK // SEGC
EPT = NCHUNK * K
EP = EPT * NS
PADN = EP - EH
SROWS = 10240
CROWS = N + 16
CFLAT = CROWS * T
CPT = CFLAT // NS
NT = N * T
CO = NT // NS
BPT = BATCH // (NC * NS)

_mesh = plsc.VectorSubcoreMesh(
    core_axis_name="c", subcore_axis_name="s", num_cores=NC, num_subcores=NS)

_f32 = jnp.float32


NDEG = SROWS


@functools.partial(
    pl.kernel,
    out_type=jax.ShapeDtypeStruct((NC * NDEG,), _f32),
    mesh=_mesh,
    scratch_types=[
        pltpu.VMEM((NCHUNK, K), jnp.int32),
        pltpu.VMEM((K,), _f32),
        pltpu.VMEM((NDEG // NS,), _f32),
        pltpu.VMEM_SHARED((NDEG,), _f32),
    ],
)
def _sc_degree(row_hbm, out_hbm, row_v, ones_v, zb_v, acc_sh):
    c = lax.axis_index("c")
    s = lax.axis_index("s")
    pltpu.sync_copy(row_hbm.at[c, s], row_v)
    zeros = jnp.zeros((L,), _f32)
    ones = jnp.ones((L,), _f32)
    def zb(i, carry):
        zb_v[pl.ds(i * L, L)] = zeros
        return carry
    lax.fori_loop(0, NDEG // NS // L, zb, 0)
    def ob(i, carry):
        ones_v[pl.ds(i * L, L)] = ones
        return carry
    lax.fori_loop(0, K // L, ob, 0)
    pltpu.sync_copy(zb_v, acc_sh.at[pl.ds(s * (NDEG // NS), NDEG // NS)])
    plsc.subcore_barrier()
    def body(j, carry):
        pltpu.sync_copy(ones_v, acc_sh.at[row_v.at[j]], add=True)
        return carry
    lax.fori_loop(0, NCHUNK, body, 0)
    plsc.subcore_barrier()
    pltpu.sync_copy(acc_sh.at[pl.ds(s * (NDEG // NS), NDEG // NS)], zb_v)
    pltpu.sync_copy(zb_v,
                    out_hbm.at[pl.ds(c * NDEG + s * (NDEG // NS), NDEG // NS)])


NPASS = 2
PROWS = N // NPASS
CDUMP = PROWS * T
CFH = CDUMP + 1600
CTILE = CFH // NS
CCH = 2504
SCAP = 15


@functools.partial(
    pl.kernel,
    out_type=jax.ShapeDtypeStruct((NC * NPASS * CFH,), _f32),
    mesh=_mesh,
    scratch_types=[
        pltpu.VMEM((SEGC, K), jnp.int32),
        pltpu.VMEM((SEGC, K), jnp.int32),
        pltpu.VMEM((SEGC, K), jnp.int32),
        pltpu.VMEM((SEGC, K), _f32),
        pltpu.VMEM((2512,), _f32),
        pltpu.VMEM((2512,), _f32),
        pltpu.VMEM_SHARED((CFH,), _f32),
        pltpu.SemaphoreType.DMA,
        pltpu.SemaphoreType.DMA,
    ],
)
def _sc_buildc(fidx_hbm, col_hbm, dinv_hbm, out_hbm,
               col_sv, fidx_sv, idxp_sv, val_sv, zb_v, cb_v, acc_sh,
               sem, sem2):
    c = lax.axis_index("c")
    s = lax.axis_index("s")
    zeros = jnp.zeros((L,), _f32)
    def zf(i, carry):
        zb_v[pl.ds(i * L, L)] = zeros
        return carry
    lax.fori_loop(0, 2512 // L, zf, 0)
    dinv = dinv_hbm.at[pl.ds(c * N, N)]
    G = K // L
    for p in range(NPASS):
        def zc(i, carry):
            pltpu.sync_copy(zb_v.at[pl.ds(0, CCH)],
                            acc_sh.at[pl.ds(s * CTILE + i * CCH, CCH)])
            return carry
        lax.fori_loop(0, CTILE // CCH, zc, 0)
        plsc.subcore_barrier()
        lo = p * CDUMP
        def seg(g, carry):
            pltpu.sync_copy(col_hbm.at[c, s, pl.ds(g * SEGC, SEGC)], col_sv)
            pltpu.sync_copy(fidx_hbm.at[c, s, pl.ds(g * SEGC, SEGC)], fidx_sv)
            def gat(j, carry2):
                pltpu.async_copy(dinv.at[col_sv.at[j]], val_sv.at[j], sem)
                return carry2
            lax.fori_loop(0, SEGC, gat, 0)
            def clamp(i, carry2):
                j = i // G
                o = (i % G) * L
                fi = fidx_sv[j, pl.ds(o, L)]
                inr = (fi >= lo) & (fi < lo + CDUMP)
                idxp_sv[j, pl.ds(o, L)] = jnp.where(inr, fi - lo, CDUMP)
                return carry2
            lax.fori_loop(0, SEGC * G, clamp, 0)
            def gatw(j, carry2):
                pltpu.make_async_copy(dinv.at[col_sv.at[j]], val_sv.at[j],
                                      sem).wait()
                return carry2
            lax.fori_loop(0, SEGC, gatw, 0)
            def scat(j, carry2):
                pltpu.async_copy(val_sv.at[j], acc_sh.at[idxp_sv.at[j]], sem2,
                                 add=True)
                return carry2
            lax.fori_loop(0, SEGC, scat, 0)
            def scatw(j, carry2):
                pltpu.make_async_copy(val_sv.at[j], acc_sh.at[idxp_sv.at[j]],
                                      sem2).wait()
                return carry2
            lax.fori_loop(0, SEGC, scatw, 0)
            return carry
        lax.fori_loop(0, NSEG, seg, 0)
        plsc.subcore_barrier()
        obase = c * (NPASS * CFH) + p * CFH + s * CTILE
        def co(i, carry):
            bounce = cb_v.at[pl.ds(0, CCH)]
            pltpu.sync_copy(acc_sh.at[pl.ds(s * CTILE + i * CCH, CCH)], bounce)
            pltpu.sync_copy(bounce, out_hbm.at[pl.ds(obase + i * CCH, CCH)])
            return carry
        lax.fori_loop(0, CTILE // CCH, co, 0)
        plsc.subcore_barrier()


@functools.partial(
    pl.kernel,
    out_type=jax.ShapeDtypeStruct((NC, SROWS, D), _f32),
    mesh=_mesh,
    scratch_types=[
        pltpu.VMEM((NCHUNK, K), jnp.int32),
        pltpu.VMEM((NCHUNK, K), jnp.int32),
        pltpu.VMEM((K, D), _f32),
        pltpu.VMEM_SHARED((SROWS, D), _f32),
        pltpu.SemaphoreType.DMA,
    ],
)
def _sc_spmm(zeros_hbm, xs_hbm, col_hbm, row_hbm, out_hbm,
             col_v, row_v, gb_v, acc_sh, sem):
    c = lax.axis_index("c")
    s = lax.axis_index("s")
    pltpu.sync_copy(col_hbm.at[c, s], col_v)
    pltpu.sync_copy(row_hbm.at[c, s], row_v)
    RPT = SROWS // NS
    pltpu.sync_copy(zeros_hbm, acc_sh.at[pl.ds(s * RPT, RPT)])
    plsc.subcore_barrier()
    xs = xs_hbm.at[c]
    def body(j, carry):
        pltpu.async_copy(xs.at[col_v.at[j]], gb_v, sem).wait()
        pltpu.sync_copy(gb_v, acc_sh.at[row_v.at[j]], add=True)
        return carry
    lax.fori_loop(0, NCHUNK, body, 0)
    plsc.subcore_barrier()
    pltpu.sync_copy(acc_sh.at[pl.ds(s * RPT, RPT)],
                    out_hbm.at[c, pl.ds(s * RPT, RPT)])


@functools.partial(
    pl.kernel,
    out_type=(jax.ShapeDtypeStruct((BATCH, D), _f32),
              jax.ShapeDtypeStruct((BATCH, D), _f32)),
    mesh=_mesh,
    scratch_types=[
        pltpu.VMEM((BPT,), jnp.int32),
        pltpu.VMEM((BPT, D), _f32),
        pltpu.SemaphoreType.DMA,
    ],
)
def _sc_gather(x_hbm, r_hbm, sub_hbm, rel_hbm, sube_hbm, rele_hbm,
               idx_v, buf_v, sem):
    c = lax.axis_index("c")
    s = lax.axis_index("s")
    base = (s * NC + c) * BPT
    pltpu.sync_copy(sub_hbm.at[pl.ds(base, BPT)], idx_v)
    pltpu.async_copy(x_hbm.at[idx_v], buf_v, sem).wait()
    pltpu.sync_copy(buf_v, sube_hbm.at[pl.ds(base, BPT)])
    pltpu.sync_copy(rel_hbm.at[pl.ds(base, BPT)], idx_v)
    pltpu.async_copy(r_hbm.at[idx_v], buf_v, sem).wait()
    pltpu.sync_copy(buf_v, rele_hbm.at[pl.ds(base, BPT)])


def _tc_pre_body(degp_ref, x_ref, dinv_ref, xs_ref):
    deg = degp_ref[:, :N]
    dinv = jnp.where(deg > 0, lax.rsqrt(jnp.maximum(deg, 1e-12)), 0.0)
    dinv_ref[...] = dinv
    xs_ref[...] = dinv[:, :, None] * x_ref[...][None, :, :]


def _tc_pre(degp, x):
    return pl.pallas_call(
        _tc_pre_body,
        out_shape=(jax.ShapeDtypeStruct((NC, N), _f32),
                   jax.ShapeDtypeStruct((NC, N, D), _f32)),
    )(degp, x)


def _tc_scale_body(dinv_ref, x_ref, xs_ref):
    xs_ref[...] = dinv_ref[...][:, :, None] * x_ref[...][None, :, :]


def _tc_scale(dinv, x):
    return pl.pallas_call(
        _tc_scale_body,
        out_shape=jax.ShapeDtypeStruct((NC, N, D), _f32),
    )(dinv, x)


def _tc_layer_body(want_xsn, x_ref, S_ref, C_ref, dinv_ref, relf_ref,
                   wl_ref, wi_ref, wo_ref, wr_ref, b_ref, g_ref, be_ref,
                   *out_refs):
    x = x_ref[...]
    relf = relf_ref[...]
    rel200 = relf[:T, :]
    dinv = dinv_ref[...]
    res = jnp.zeros((N, D), _f32)
    for h, w_ref in ((0, wi_ref), (1, wo_ref)):
        Rh = jnp.dot(C_ref[h], rel200, preferred_element_type=_f32)
        agg = dinv[h][:, None] * (S_ref[h] - Rh)
        res = res + jnp.dot(agg, w_ref[...], preferred_element_type=_f32)
    loop_res = jnp.dot(x - relf[T, :][None, :], wl_ref[...],
                       preferred_element_type=_f32)
    out = (res + loop_res) * (1.0 / 3.0) + b_ref[...][None, :]
    mean = jnp.mean(out, axis=0)
    var = jnp.mean((out - mean[None, :]) ** 2, axis=0)
    out = (out - mean[None, :]) * lax.rsqrt(var + 1e-5)[None, :] * \
        g_ref[...][None, :] + be_ref[...][None, :]
    out = jnp.tanh(out)
    out_refs[0][...] = out
    out_refs[1][...] = jnp.dot(relf, wr_ref[...],
                               preferred_element_type=_f32)[:T, :]
    if want_xsn:
        out_refs[2][...] = dinv[:, :, None] * out[None, :, :]


def _tc_layer(x, S, C, dinv, relf, wl, wi, wo, wr, b, g, be, want_xsn):
    outs = [jax.ShapeDtypeStruct((N, D), _f32),
            jax.ShapeDtypeStruct((T, D), _f32)]
    if want_xsn:
        outs.append(jax.ShapeDtypeStruct((NC, N, D), _f32))
    return pl.pallas_call(
        functools.partial(_tc_layer_body, want_xsn),
        out_shape=tuple(outs),
    )(x, S, C, dinv, relf, wl, wi, wo, wr, b, g, be)


def kernel(sub, rel, edge_index, edge_type, init_embed, init_rel,
           w_loop1, w_in1, w_out1, w_rel1, loop_rel1, bias1, gamma1, beta1,
           w_loop2, w_in2, w_out2, w_rel2, loop_rel2, bias2, gamma2, beta2):
    ei = edge_index.astype(jnp.int32)
    ety = edge_type.astype(jnp.int32)
    pad_row = jnp.full((PADN,), N, jnp.int32)
    pad_zero = jnp.zeros((PADN,), jnp.int32)
    row_p4 = jnp.stack([jnp.concatenate([ei[0, :EH], pad_row]),
                        jnp.concatenate([ei[0, EH:], pad_row])]
                       ).reshape(NC, NS, NCHUNK, K)
    col_p4 = jnp.stack([jnp.concatenate([ei[1, :EH], pad_zero]),
                        jnp.concatenate([ei[1, EH:], pad_zero])]
                       ).reshape(NC, NS, NCHUNK, K)
    et_p4 = jnp.stack([jnp.concatenate([ety[:EH], pad_zero]),
                       jnp.concatenate([ety[EH:], pad_zero])]
                      ).reshape(NC, NS, NCHUNK, K)
    fidx_p4 = row_p4 * T + et_p4

    degp = _sc_degree(row_p4).reshape(NC, NDEG)
    dinv, xs1 = _tc_pre(degp, init_embed)
    Craw = _sc_buildc(fidx_p4, col_p4, dinv.reshape(-1))
    C = Craw.reshape(NC, NPASS, CFH)[:, :, :CDUMP].reshape(NC, N, T)

    zrows = jnp.zeros((SROWS // NS, D), _f32)
    S1 = _sc_spmm(zrows, xs1, col_p4, row_p4)[:, :N, :]
    relf1 = jnp.concatenate([init_rel, loop_rel1], axis=0)
    x1, r1 = _tc_layer(init_embed, S1, C, dinv, relf1,
                       w_loop1, w_in1, w_out1, w_rel1,
                       bias1, gamma1, beta1, False)
    xs2 = _tc_scale(dinv, x1)

    S2 = _sc_spmm(zrows, xs2, col_p4, row_p4)[:, :N, :]
    relf2 = jnp.concatenate([r1, loop_rel2], axis=0)
    x2, r2 = _tc_layer(x1, S2, C, dinv, relf2,
                       w_loop2, w_in2, w_out2, w_rel2,
                       bias2, gamma2, beta2, False)

    sub_emb, rel_emb = _sc_gather(x2, r2, sub.astype(jnp.int32),
                                  rel.astype(jnp.int32))
    return sub_emb, rel_emb, x2


# speedup vs baseline: 1.8165x; 1.0831x over previous
import functools

import jax
import jax.numpy as jnp
from jax import lax
from jax.experimental import pallas as pl
from jax.experimental.pallas import tpu as pltpu
from jax.experimental.pallas import tpu_sc as plsc

N = 10000
D = 128
T = 200
BATCH = 1024
NC, NS, L = 2, 16, 16
EH = 160000
K = 128
NCHUNK = 80
SEGC = 16
NSEG = NCHUNK // SEGC
EPT = NCHUNK * K
EP = EPT * NS
PADN = EP - EH
SROWS = 10240
CROWS = N + 16
CFLAT = CROWS * T
CPT = CFLAT // NS
NT = N * T
CO = NT // NS
BPT = BATCH // (NC * NS)

_mesh = plsc.VectorSubcoreMesh(
    core_axis_name="c", subcore_axis_name="s", num_cores=NC, num_subcores=NS)

_f32 = jnp.float32


NDEG = SROWS


@functools.partial(
    pl.kernel,
    out_type=jax.ShapeDtypeStruct((NC * NDEG,), _f32),
    mesh=_mesh,
    scratch_types=[
        pltpu.VMEM((NCHUNK, K), jnp.int32),
        pltpu.VMEM((K,), _f32),
        pltpu.VMEM((NDEG // NS,), _f32),
        pltpu.VMEM_SHARED((NDEG,), _f32),
    ],
)
def _sc_degree(row_hbm, out_hbm, row_v, ones_v, zb_v, acc_sh):
    c = lax.axis_index("c")
    s = lax.axis_index("s")
    pltpu.sync_copy(row_hbm.at[c, s], row_v)
    zeros = jnp.zeros((L,), _f32)
    ones = jnp.ones((L,), _f32)
    def zb(i, carry):
        zb_v[pl.ds(i * L, L)] = zeros
        return carry
    lax.fori_loop(0, NDEG // NS // L, zb, 0)
    def ob(i, carry):
        ones_v[pl.ds(i * L, L)] = ones
        return carry
    lax.fori_loop(0, K // L, ob, 0)
    pltpu.sync_copy(zb_v, acc_sh.at[pl.ds(s * (NDEG // NS), NDEG // NS)])
    plsc.subcore_barrier()
    def body(j, carry):
        pltpu.sync_copy(ones_v, acc_sh.at[row_v.at[j]], add=True)
        return carry
    lax.fori_loop(0, NCHUNK, body, 0)
    plsc.subcore_barrier()
    pltpu.sync_copy(acc_sh.at[pl.ds(s * (NDEG // NS), NDEG // NS)], zb_v)
    pltpu.sync_copy(zb_v,
                    out_hbm.at[pl.ds(c * NDEG + s * (NDEG // NS), NDEG // NS)])


NPASS = 2
PROWS = N // NPASS
CDUMP = PROWS * T
CFH = CDUMP + 1600
CTILE = CFH // NS
CCH = 2504
SCAP = 15


@functools.partial(
    pl.kernel,
    out_type=jax.ShapeDtypeStruct((NC * NPASS * CFH,), _f32),
    mesh=_mesh,
    scratch_types=[
        pltpu.VMEM((SEGC, K), jnp.int32),
        pltpu.VMEM((SEGC, K), jnp.int32),
        pltpu.VMEM((SEGC, K), jnp.int32),
        pltpu.VMEM((SEGC, K), _f32),
        pltpu.VMEM((2512,), _f32),
        pltpu.VMEM((2512,), _f32),
        pltpu.VMEM_SHARED((CFH,), _f32),
        pltpu.SemaphoreType.DMA,
        pltpu.SemaphoreType.DMA,
    ],
)
def _sc_buildc(fidx_hbm, col_hbm, dinv_hbm, out_hbm,
               col_sv, fidx_sv, idxp_sv, val_sv, zb_v, cb_v, acc_sh,
               sem, sem2):
    c = lax.axis_index("c")
    s = lax.axis_index("s")
    zeros = jnp.zeros((L,), _f32)
    def zf(i, carry):
        zb_v[pl.ds(i * L, L)] = zeros
        return carry
    lax.fori_loop(0, 2512 // L, zf, 0)
    dinv = dinv_hbm.at[pl.ds(c * N, N)]
    G = K // L
    for p in range(NPASS):
        def zc(i, carry):
            pltpu.sync_copy(zb_v.at[pl.ds(0, CCH)],
                            acc_sh.at[pl.ds(s * CTILE + i * CCH, CCH)])
            return carry
        lax.fori_loop(0, CTILE // CCH, zc, 0)
        plsc.subcore_barrier()
        lo = p * CDUMP
        def seg(g, carry):
            pltpu.sync_copy(col_hbm.at[c, s, pl.ds(g * SEGC, SEGC)], col_sv)
            pltpu.sync_copy(fidx_hbm.at[c, s, pl.ds(g * SEGC, SEGC)], fidx_sv)
            def gat(j, carry2):
                pltpu.async_copy(dinv.at[col_sv.at[j]], val_sv.at[j], sem)
                return carry2
            lax.fori_loop(0, SEGC, gat, 0)
            def clamp(i, carry2):
                j = i // G
                o = (i % G) * L
                fi = fidx_sv[j, pl.ds(o, L)]
                inr = (fi >= lo) & (fi < lo + CDUMP)
                idxp_sv[j, pl.ds(o, L)] = jnp.where(inr, fi - lo, CDUMP)
                return carry2
            lax.fori_loop(0, SEGC * G, clamp, 0)
            def gatw(j, carry2):
                pltpu.make_async_copy(dinv.at[col_sv.at[j]], val_sv.at[j],
                                      sem).wait()
                return carry2
            lax.fori_loop(0, SEGC, gatw, 0)
            def scat(j, carry2):
                pltpu.async_copy(val_sv.at[j], acc_sh.at[idxp_sv.at[j]], sem2,
                                 add=True)
                return carry2
            lax.fori_loop(0, SEGC, scat, 0)
            def scatw(j, carry2):
                pltpu.make_async_copy(val_sv.at[j], acc_sh.at[idxp_sv.at[j]],
                                      sem2).wait()
                return carry2
            lax.fori_loop(0, SEGC, scatw, 0)
            return carry
        lax.fori_loop(0, NSEG, seg, 0)
        plsc.subcore_barrier()
        obase = c * (NPASS * CFH) + p * CFH + s * CTILE
        def co(i, carry):
            bounce = cb_v.at[pl.ds(0, CCH)]
            pltpu.sync_copy(acc_sh.at[pl.ds(s * CTILE + i * CCH, CCH)], bounce)
            pltpu.sync_copy(bounce, out_hbm.at[pl.ds(obase + i * CCH, CCH)])
            return carry
        lax.fori_loop(0, CTILE // CCH, co, 0)
        plsc.subcore_barrier()


@functools.partial(
    pl.kernel,
    out_type=jax.ShapeDtypeStruct((NC, SROWS, D), _f32),
    mesh=_mesh,
    scratch_types=[
        pltpu.VMEM((NCHUNK, K), jnp.int32),
        pltpu.VMEM((NCHUNK, K), jnp.int32),
        pltpu.VMEM((K, D), _f32),
        pltpu.VMEM_SHARED((SROWS, D), _f32),
        pltpu.SemaphoreType.DMA,
    ],
)
def _sc_spmm(zeros_hbm, xs_hbm, col_hbm, row_hbm, out_hbm,
             col_v, row_v, gb_v, acc_sh, sem):
    c = lax.axis_index("c")
    s = lax.axis_index("s")
    pltpu.sync_copy(col_hbm.at[c, s], col_v)
    pltpu.sync_copy(row_hbm.at[c, s], row_v)
    RPT = SROWS // NS
    pltpu.sync_copy(zeros_hbm, acc_sh.at[pl.ds(s * RPT, RPT)])
    plsc.subcore_barrier()
    xs = xs_hbm.at[c]
    def body(j, carry):
        pltpu.async_copy(xs.at[col_v.at[j]], gb_v, sem).wait()
        pltpu.sync_copy(gb_v, acc_sh.at[row_v.at[j]], add=True)
        return carry
    lax.fori_loop(0, NCHUNK, body, 0)
    plsc.subcore_barrier()
    pltpu.sync_copy(acc_sh.at[pl.ds(s * RPT, RPT)],
                    out_hbm.at[c, pl.ds(s * RPT, RPT)])


@functools.partial(
    pl.kernel,
    out_type=(jax.ShapeDtypeStruct((BATCH, D), _f32),
              jax.ShapeDtypeStruct((BATCH, D), _f32)),
    mesh=_mesh,
    scratch_types=[
        pltpu.VMEM((BPT,), jnp.int32),
        pltpu.VMEM((BPT, D), _f32),
        pltpu.SemaphoreType.DMA,
    ],
)
def _sc_gather(x_hbm, r_hbm, sub_hbm, rel_hbm, sube_hbm, rele_hbm,
               idx_v, buf_v, sem):
    c = lax.axis_index("c")
    s = lax.axis_index("s")
    base = (s * NC + c) * BPT
    pltpu.sync_copy(sub_hbm.at[pl.ds(base, BPT)], idx_v)
    pltpu.async_copy(x_hbm.at[idx_v], buf_v, sem).wait()
    pltpu.sync_copy(buf_v, sube_hbm.at[pl.ds(base, BPT)])
    pltpu.sync_copy(rel_hbm.at[pl.ds(base, BPT)], idx_v)
    pltpu.async_copy(r_hbm.at[idx_v], buf_v, sem).wait()
    pltpu.sync_copy(buf_v, rele_hbm.at[pl.ds(base, BPT)])


def _tc_pre_body(degp_ref, x_ref, dinv_ref, xs_ref):
    deg = degp_ref[:, :N]
    dinv = jnp.where(deg > 0, lax.rsqrt(jnp.maximum(deg, 1e-12)), 0.0)
    dinv_ref[...] = dinv
    xs_ref[...] = dinv[:, :, None] * x_ref[...][None, :, :]


def _tc_pre(degp, x):
    return pl.pallas_call(
        _tc_pre_body,
        out_shape=(jax.ShapeDtypeStruct((NC, N), _f32),
                   jax.ShapeDtypeStruct((NC, N, D), _f32)),
    )(degp, x)


def _tc_scale_body(dinv_ref, x_ref, xs_ref):
    xs_ref[...] = dinv_ref[...][:, :, None] * x_ref[...][None, :, :]


def _tc_scale(dinv, x):
    return pl.pallas_call(
        _tc_scale_body,
        out_shape=jax.ShapeDtypeStruct((NC, N, D), _f32),
    )(dinv, x)


RB = 2000


def _tc_aggmm_body(x_ref, S_ref, C_ref, dinv_ref, relf_ref,
                   wl_ref, wi_ref, wo_ref, b_ref, pre_ref):
    x = x_ref[...]
    relf = relf_ref[...]
    rel200 = relf[:T, :]
    dinv = dinv_ref[0]
    res = jnp.zeros((RB, D), _f32)
    for h, w_ref in ((0, wi_ref), (1, wo_ref)):
        Rh = jnp.dot(C_ref[h], rel200, preferred_element_type=_f32)
        agg = dinv[h][:, None] * (S_ref[h] - Rh)
        res = res + jnp.dot(agg, w_ref[...], preferred_element_type=_f32)
    loop_res = jnp.dot(x - relf[T, :][None, :], wl_ref[...],
                       preferred_element_type=_f32)
    pre_ref[...] = (res + loop_res) * (1.0 / 3.0) + b_ref[...][None, :]


def _tc_bn_body(pre_ref, relf_ref, wr_ref, g_ref, be_ref, out_ref, nrel_ref):
    out = pre_ref[...]
    mean = jnp.mean(out, axis=0)
    var = jnp.mean((out - mean[None, :]) ** 2, axis=0)
    out = (out - mean[None, :]) * lax.rsqrt(var + 1e-5)[None, :] * \
        g_ref[...][None, :] + be_ref[...][None, :]
    out_ref[...] = jnp.tanh(out)
    nrel_ref[...] = jnp.dot(relf_ref[...], wr_ref[...],
                            preferred_element_type=_f32)[:T, :]


def _tc_layer(x, S, C, dinv, relf, wl, wi, wo, wr, b, g, be, want_xsn):
    del want_xsn
    full = lambda *shape: pl.BlockSpec(shape, lambda i: (0,) * len(shape))
    pre = pl.pallas_call(
        _tc_aggmm_body,
        grid=(N // RB,),
        in_specs=[
            pl.BlockSpec((RB, D), lambda i: (i, 0)),
            pl.BlockSpec((NC, RB, D), lambda i: (0, i, 0)),
            pl.BlockSpec((NC, RB, T), lambda i: (0, i, 0)),
            pl.BlockSpec((1, NC, RB), lambda i: (i, 0, 0)),
            full(T + 1, D),
            full(D, D),
            full(D, D),
            full(D, D),
            full(D),
        ],
        out_specs=pl.BlockSpec((RB, D), lambda i: (i, 0)),
        out_shape=jax.ShapeDtypeStruct((N, D), _f32),
    )(x, S, C, dinv.reshape(NC, N // RB, RB).transpose(1, 0, 2),
      relf, wl, wi, wo, b)
    return pl.pallas_call(
        _tc_bn_body,
        out_shape=(jax.ShapeDtypeStruct((N, D), _f32),
                   jax.ShapeDtypeStruct((T, D), _f32)),
    )(pre, relf, wr, g, be)


def kernel(sub, rel, edge_index, edge_type, init_embed, init_rel,
           w_loop1, w_in1, w_out1, w_rel1, loop_rel1, bias1, gamma1, beta1,
           w_loop2, w_in2, w_out2, w_rel2, loop_rel2, bias2, gamma2, beta2):
    ei = edge_index.astype(jnp.int32)
    ety = edge_type.astype(jnp.int32)
    pad_row = jnp.full((PADN,), N, jnp.int32)
    pad_zero = jnp.zeros((PADN,), jnp.int32)
    row_p4 = jnp.stack([jnp.concatenate([ei[0, :EH], pad_row]),
                        jnp.concatenate([ei[0, EH:], pad_row])]
                       ).reshape(NC, NS, NCHUNK, K)
    col_p4 = jnp.stack([jnp.concatenate([ei[1, :EH], pad_zero]),
                        jnp.concatenate([ei[1, EH:], pad_zero])]
                       ).reshape(NC, NS, NCHUNK, K)
    et_p4 = jnp.stack([jnp.concatenate([ety[:EH], pad_zero]),
                       jnp.concatenate([ety[EH:], pad_zero])]
                      ).reshape(NC, NS, NCHUNK, K)
    fidx_p4 = row_p4 * T + et_p4

    degp = _sc_degree(row_p4).reshape(NC, NDEG)
    dinv, xs1 = _tc_pre(degp, init_embed)
    Craw = _sc_buildc(fidx_p4, col_p4, dinv.reshape(-1))
    C = Craw.reshape(NC, NPASS, CFH)[:, :, :CDUMP].reshape(NC, N, T)

    zrows = jnp.zeros((SROWS // NS, D), _f32)
    S1 = _sc_spmm(zrows, xs1, col_p4, row_p4)[:, :N, :]
    relf1 = jnp.concatenate([init_rel, loop_rel1], axis=0)
    x1, r1 = _tc_layer(init_embed, S1, C, dinv, relf1,
                       w_loop1, w_in1, w_out1, w_rel1,
                       bias1, gamma1, beta1, False)
    xs2 = _tc_scale(dinv, x1)

    S2 = _sc_spmm(zrows, xs2, col_p4, row_p4)[:, :N, :]
    relf2 = jnp.concatenate([r1, loop_rel2], axis=0)
    x2, r2 = _tc_layer(x1, S2, C, dinv, relf2,
                       w_loop2, w_in2, w_out2, w_rel2,
                       bias2, gamma2, beta2, False)

    sub_emb, rel_emb = _sc_gather(x2, r2, sub.astype(jnp.int32),
                                  rel.astype(jnp.int32))
    return sub_emb, rel_emb, x2


# speedup vs baseline: 1.8217x; 1.0028x over previous
import functools

import jax
import jax.numpy as jnp
from jax import lax
from jax.experimental import pallas as pl
from jax.experimental.pallas import tpu as pltpu
from jax.experimental.pallas import tpu_sc as plsc

N = 10000
D = 128
T = 200
BATCH = 1024
NC, NS, L = 2, 16, 16
EH = 160000
K = 128
NCHUNK = 80
SEGC = 16
NSEG = NCHUNK // SEGC
EPT = NCHUNK * K
EP = EPT * NS
PADN = EP - EH
SROWS = 10240
CROWS = N + 16
CFLAT = CROWS * T
CPT = CFLAT // NS
NT = N * T
CO = NT // NS
BPT = BATCH // (NC * NS)

_mesh = plsc.VectorSubcoreMesh(
    core_axis_name="c", subcore_axis_name="s", num_cores=NC, num_subcores=NS)

_f32 = jnp.float32


NDEG = SROWS


@functools.partial(
    pl.kernel,
    out_type=jax.ShapeDtypeStruct((NC * NDEG,), _f32),
    mesh=_mesh,
    scratch_types=[
        pltpu.VMEM((NCHUNK, K), jnp.int32),
        pltpu.VMEM((K,), _f32),
        pltpu.VMEM((NDEG // NS,), _f32),
        pltpu.VMEM_SHARED((NDEG,), _f32),
    ],
)
def _sc_degree(row_hbm, out_hbm, row_v, ones_v, zb_v, acc_sh):
    c = lax.axis_index("c")
    s = lax.axis_index("s")
    pltpu.sync_copy(row_hbm.at[c, s], row_v)
    zeros = jnp.zeros((L,), _f32)
    ones = jnp.ones((L,), _f32)
    def zb(i, carry):
        zb_v[pl.ds(i * L, L)] = zeros
        return carry
    lax.fori_loop(0, NDEG // NS // L, zb, 0)
    def ob(i, carry):
        ones_v[pl.ds(i * L, L)] = ones
        return carry
    lax.fori_loop(0, K // L, ob, 0)
    pltpu.sync_copy(zb_v, acc_sh.at[pl.ds(s * (NDEG // NS), NDEG // NS)])
    plsc.subcore_barrier()
    def body(j, carry):
        pltpu.sync_copy(ones_v, acc_sh.at[row_v.at[j]], add=True)
        return carry
    lax.fori_loop(0, NCHUNK, body, 0)
    plsc.subcore_barrier()
    pltpu.sync_copy(acc_sh.at[pl.ds(s * (NDEG // NS), NDEG // NS)], zb_v)
    pltpu.sync_copy(zb_v,
                    out_hbm.at[pl.ds(c * NDEG + s * (NDEG // NS), NDEG // NS)])


NPASS = 2
PROWS = N // NPASS
CDUMP = PROWS * T
CFH = CDUMP + 1600
CTILE = CFH // NS
CCH = 2504
SCAP = 15


@functools.partial(
    pl.kernel,
    out_type=jax.ShapeDtypeStruct((NC * NPASS * CFH,), _f32),
    mesh=_mesh,
    scratch_types=[
        pltpu.VMEM((SEGC, K), jnp.int32),
        pltpu.VMEM((SEGC, K), jnp.int32),
        pltpu.VMEM((SEGC, K), jnp.int32),
        pltpu.VMEM((SEGC, K), _f32),
        pltpu.VMEM((2512,), _f32),
        pltpu.VMEM((2512,), _f32),
        pltpu.VMEM_SHARED((CFH,), _f32),
        pltpu.SemaphoreType.DMA,
        pltpu.SemaphoreType.DMA,
    ],
)
def _sc_buildc(fidx_hbm, col_hbm, dinv_hbm, out_hbm,
               col_sv, fidx_sv, idxp_sv, val_sv, zb_v, cb_v, acc_sh,
               sem, sem2):
    c = lax.axis_index("c")
    s = lax.axis_index("s")
    zeros = jnp.zeros((L,), _f32)
    def zf(i, carry):
        zb_v[pl.ds(i * L, L)] = zeros
        return carry
    lax.fori_loop(0, 2512 // L, zf, 0)
    dinv = dinv_hbm.at[pl.ds(c * N, N)]
    G = K // L
    for p in range(NPASS):
        def zc(i, carry):
            pltpu.sync_copy(zb_v.at[pl.ds(0, CCH)],
                            acc_sh.at[pl.ds(s * CTILE + i * CCH, CCH)])
            return carry
        lax.fori_loop(0, CTILE // CCH, zc, 0)
        plsc.subcore_barrier()
        lo = p * CDUMP
        def seg(g, carry):
            pltpu.sync_copy(col_hbm.at[c, s, pl.ds(g * SEGC, SEGC)], col_sv)
            pltpu.sync_copy(fidx_hbm.at[c, s, pl.ds(g * SEGC, SEGC)], fidx_sv)
            def gat(j, carry2):
                pltpu.async_copy(dinv.at[col_sv.at[j]], val_sv.at[j], sem)
                return carry2
            lax.fori_loop(0, SEGC, gat, 0)
            def clamp(i, carry2):
                j = i // G
                o = (i % G) * L
                fi = fidx_sv[j, pl.ds(o, L)]
                inr = (fi >= lo) & (fi < lo + CDUMP)
                idxp_sv[j, pl.ds(o, L)] = jnp.where(inr, fi - lo, CDUMP)
                return carry2
            lax.fori_loop(0, SEGC * G, clamp, 0)
            def gatw(j, carry2):
                pltpu.make_async_copy(dinv.at[col_sv.at[j]], val_sv.at[j],
                                      sem).wait()
                return carry2
            lax.fori_loop(0, SEGC, gatw, 0)
            def scat(j, carry2):
                pltpu.async_copy(val_sv.at[j], acc_sh.at[idxp_sv.at[j]], sem2,
                                 add=True)
                return carry2
            lax.fori_loop(0, SEGC, scat, 0)
            def scatw(j, carry2):
                pltpu.make_async_copy(val_sv.at[j], acc_sh.at[idxp_sv.at[j]],
                                      sem2).wait()
                return carry2
            lax.fori_loop(0, SEGC, scatw, 0)
            return carry
        lax.fori_loop(0, NSEG, seg, 0)
        plsc.subcore_barrier()
        obase = c * (NPASS * CFH) + p * CFH + s * CTILE
        def co(i, carry):
            bounce = cb_v.at[pl.ds(0, CCH)]
            pltpu.sync_copy(acc_sh.at[pl.ds(s * CTILE + i * CCH, CCH)], bounce)
            pltpu.sync_copy(bounce, out_hbm.at[pl.ds(obase + i * CCH, CCH)])
            return carry
        lax.fori_loop(0, CTILE // CCH, co, 0)
        plsc.subcore_barrier()


@functools.partial(
    pl.kernel,
    out_type=jax.ShapeDtypeStruct((NC, SROWS, D), _f32),
    mesh=_mesh,
    scratch_types=[
        pltpu.VMEM((NCHUNK, K), jnp.int32),
        pltpu.VMEM((NCHUNK, K), jnp.int32),
        pltpu.VMEM((K, D), _f32),
        pltpu.VMEM_SHARED((SROWS, D), _f32),
        pltpu.SemaphoreType.DMA,
    ],
)
def _sc_spmm(zeros_hbm, xs_hbm, col_hbm, row_hbm, out_hbm,
             col_v, row_v, gb_v, acc_sh, sem):
    c = lax.axis_index("c")
    s = lax.axis_index("s")
    pltpu.sync_copy(col_hbm.at[c, s], col_v)
    pltpu.sync_copy(row_hbm.at[c, s], row_v)
    RL = D // L
    RPT = SROWS // NS
    zeros = jnp.zeros((L,), _f32)
    def zb(i, carry):
        gb_v[i // RL, pl.ds((i % RL) * L, L)] = zeros
        return carry
    lax.fori_loop(0, K * RL, zb, 0)
    def zc(i, carry):
        pltpu.sync_copy(gb_v, acc_sh.at[pl.ds(s * RPT + i * K, K)])
        return carry
    lax.fori_loop(0, RPT // K, zc, 0)
    plsc.subcore_barrier()
    xs = xs_hbm.at[c]
    def body(j, carry):
        pltpu.async_copy(xs.at[col_v.at[j]], gb_v, sem).wait()
        pltpu.sync_copy(gb_v, acc_sh.at[row_v.at[j]], add=True)
        return carry
    lax.fori_loop(0, NCHUNK, body, 0)
    plsc.subcore_barrier()
    def co(i, carry):
        pltpu.sync_copy(acc_sh.at[pl.ds(s * RPT + i * K, K)], gb_v)
        pltpu.sync_copy(gb_v, out_hbm.at[c, pl.ds(s * RPT + i * K, K)])
        return carry
    lax.fori_loop(0, RPT // K, co, 0)


@functools.partial(
    pl.kernel,
    out_type=(jax.ShapeDtypeStruct((BATCH, D), _f32),
              jax.ShapeDtypeStruct((BATCH, D), _f32)),
    mesh=_mesh,
    scratch_types=[
        pltpu.VMEM((BPT,), jnp.int32),
        pltpu.VMEM((BPT, D), _f32),
        pltpu.SemaphoreType.DMA,
    ],
)
def _sc_gather(x_hbm, r_hbm, sub_hbm, rel_hbm, sube_hbm, rele_hbm,
               idx_v, buf_v, sem):
    c = lax.axis_index("c")
    s = lax.axis_index("s")
    base = (s * NC + c) * BPT
    pltpu.sync_copy(sub_hbm.at[pl.ds(base, BPT)], idx_v)
    pltpu.async_copy(x_hbm.at[idx_v], buf_v, sem).wait()
    pltpu.sync_copy(buf_v, sube_hbm.at[pl.ds(base, BPT)])
    pltpu.sync_copy(rel_hbm.at[pl.ds(base, BPT)], idx_v)
    pltpu.async_copy(r_hbm.at[idx_v], buf_v, sem).wait()
    pltpu.sync_copy(buf_v, rele_hbm.at[pl.ds(base, BPT)])


def _tc_pre_body(degp_ref, x_ref, dinv_ref, xs_ref):
    deg = degp_ref[:, :N]
    dinv = jnp.where(deg > 0, lax.rsqrt(jnp.maximum(deg, 1e-12)), 0.0)
    dinv_ref[...] = dinv
    xs_ref[...] = dinv[:, :, None] * x_ref[...][None, :, :]


def _tc_pre(degp, x):
    return pl.pallas_call(
        _tc_pre_body,
        out_shape=(jax.ShapeDtypeStruct((NC, N), _f32),
                   jax.ShapeDtypeStruct((NC, N, D), _f32)),
    )(degp, x)


def _tc_scale_body(dinv_ref, x_ref, xs_ref):
    xs_ref[...] = dinv_ref[...][:, :, None] * x_ref[...][None, :, :]


def _tc_scale(dinv, x):
    return pl.pallas_call(
        _tc_scale_body,
        out_shape=jax.ShapeDtypeStruct((NC, N, D), _f32),
    )(dinv, x)


RB = 2000


def _tc_aggmm_body(x_ref, S_ref, C_ref, dinv_ref, relf_ref,
                   wl_ref, wi_ref, wo_ref, b_ref, pre_ref):
    x = x_ref[...]
    relf = relf_ref[...]
    rel200 = relf[:T, :]
    dinv = dinv_ref[0]
    res = jnp.zeros((RB, D), _f32)
    for h, w_ref in ((0, wi_ref), (1, wo_ref)):
        Rh = jnp.dot(C_ref[h], rel200, preferred_element_type=_f32)
        agg = dinv[h][:, None] * (S_ref[h] - Rh)
        res = res + jnp.dot(agg, w_ref[...], preferred_element_type=_f32)
    loop_res = jnp.dot(x - relf[T, :][None, :], wl_ref[...],
                       preferred_element_type=_f32)
    pre_ref[...] = (res + loop_res) * (1.0 / 3.0) + b_ref[...][None, :]


def _tc_bn_body(pre_ref, relf_ref, wr_ref, g_ref, be_ref, out_ref, nrel_ref):
    out = pre_ref[...]
    mean = jnp.mean(out, axis=0)
    var = jnp.mean((out - mean[None, :]) ** 2, axis=0)
    out = (out - mean[None, :]) * lax.rsqrt(var + 1e-5)[None, :] * \
        g_ref[...][None, :] + be_ref[...][None, :]
    out_ref[...] = jnp.tanh(out)
    nrel_ref[...] = jnp.dot(relf_ref[...], wr_ref[...],
                            preferred_element_type=_f32)[:T, :]


def _tc_layer(x, S, C, dinv, relf, wl, wi, wo, wr, b, g, be, want_xsn):
    del want_xsn
    full = lambda *shape: pl.BlockSpec(shape, lambda i: (0,) * len(shape))
    pre = pl.pallas_call(
        _tc_aggmm_body,
        grid=(N // RB,),
        in_specs=[
            pl.BlockSpec((RB, D), lambda i: (i, 0)),
            pl.BlockSpec((NC, RB, D), lambda i: (0, i, 0)),
            pl.BlockSpec((NC, RB, T), lambda i: (0, i, 0)),
            pl.BlockSpec((1, NC, RB), lambda i: (i, 0, 0)),
            full(T + 1, D),
            full(D, D),
            full(D, D),
            full(D, D),
            full(D),
        ],
        out_specs=pl.BlockSpec((RB, D), lambda i: (i, 0)),
        out_shape=jax.ShapeDtypeStruct((N, D), _f32),
    )(x, S, C, dinv.reshape(NC, N // RB, RB).transpose(1, 0, 2),
      relf, wl, wi, wo, b)
    return pl.pallas_call(
        _tc_bn_body,
        out_shape=(jax.ShapeDtypeStruct((N, D), _f32),
                   jax.ShapeDtypeStruct((T, D), _f32)),
    )(pre, relf, wr, g, be)


def kernel(sub, rel, edge_index, edge_type, init_embed, init_rel,
           w_loop1, w_in1, w_out1, w_rel1, loop_rel1, bias1, gamma1, beta1,
           w_loop2, w_in2, w_out2, w_rel2, loop_rel2, bias2, gamma2, beta2):
    ei = edge_index.astype(jnp.int32)
    ety = edge_type.astype(jnp.int32)
    pad_row = jnp.full((PADN,), N, jnp.int32)
    pad_zero = jnp.zeros((PADN,), jnp.int32)
    row_p4 = jnp.stack([jnp.concatenate([ei[0, :EH], pad_row]),
                        jnp.concatenate([ei[0, EH:], pad_row])]
                       ).reshape(NC, NS, NCHUNK, K)
    col_p4 = jnp.stack([jnp.concatenate([ei[1, :EH], pad_zero]),
                        jnp.concatenate([ei[1, EH:], pad_zero])]
                       ).reshape(NC, NS, NCHUNK, K)
    et_p4 = jnp.stack([jnp.concatenate([ety[:EH], pad_zero]),
                       jnp.concatenate([ety[EH:], pad_zero])]
                      ).reshape(NC, NS, NCHUNK, K)
    fidx_p4 = row_p4 * T + et_p4

    degp = _sc_degree(row_p4).reshape(NC, NDEG)
    dinv, xs1 = _tc_pre(degp, init_embed)
    Craw = _sc_buildc(fidx_p4, col_p4, dinv.reshape(-1))
    C = Craw.reshape(NC, NPASS, CFH)[:, :, :CDUMP].reshape(NC, N, T)

    zrows = jnp.zeros((SROWS // NS, D), _f32)
    S1 = _sc_spmm(zrows, xs1, col_p4, row_p4)[:, :N, :]
    relf1 = jnp.concatenate([init_rel, loop_rel1], axis=0)
    x1, r1 = _tc_layer(init_embed, S1, C, dinv, relf1,
                       w_loop1, w_in1, w_out1, w_rel1,
                       bias1, gamma1, beta1, False)
    xs2 = _tc_scale(dinv, x1)

    S2 = _sc_spmm(zrows, xs2, col_p4, row_p4)[:, :N, :]
    relf2 = jnp.concatenate([r1, loop_rel2], axis=0)
    x2, r2 = _tc_layer(x1, S2, C, dinv, relf2,
                       w_loop2, w_in2, w_out2, w_rel2,
                       bias2, gamma2, beta2, False)

    sub_emb, rel_emb = _sc_gather(x2, r2, sub.astype(jnp.int32),
                                  rel.astype(jnp.int32))
    return sub_emb, rel_emb, x2


# speedup vs baseline: 2.0484x; 1.1245x over previous
import functools

import jax
import jax.numpy as jnp
from jax import lax
from jax.experimental import pallas as pl
from jax.experimental.pallas import tpu as pltpu
from jax.experimental.pallas import tpu_sc as plsc

N = 10000
D = 128
T = 200
BATCH = 1024
NC, NS, L = 2, 16, 16
EH = 160000
K = 128
NCHUNK = 80
SEGC = 16
NSEG = NCHUNK // SEGC
EPT = NCHUNK * K
EP = EPT * NS
PADN = EP - EH
SROWS = 10240
CROWS = N + 16
CFLAT = CROWS * T
CPT = CFLAT // NS
NT = N * T
CO = NT // NS
BPT = BATCH // (NC * NS)

_mesh = plsc.VectorSubcoreMesh(
    core_axis_name="c", subcore_axis_name="s", num_cores=NC, num_subcores=NS)

_f32 = jnp.float32


NDEG = SROWS


@functools.partial(
    pl.kernel,
    out_type=jax.ShapeDtypeStruct((NC * NDEG,), _f32),
    mesh=_mesh,
    scratch_types=[
        pltpu.VMEM((NCHUNK, K), jnp.int32),
        pltpu.VMEM((K,), _f32),
        pltpu.VMEM((NDEG // NS,), _f32),
        pltpu.VMEM_SHARED((NDEG,), _f32),
    ],
)
def _sc_degree(row_hbm, out_hbm, row_v, ones_v, zb_v, acc_sh):
    c = lax.axis_index("c")
    s = lax.axis_index("s")
    pltpu.sync_copy(row_hbm.at[c, s], row_v)
    zeros = jnp.zeros((L,), _f32)
    ones = jnp.ones((L,), _f32)
    def zb(i, carry):
        zb_v[pl.ds(i * L, L)] = zeros
        return carry
    lax.fori_loop(0, NDEG // NS // L, zb, 0)
    def ob(i, carry):
        ones_v[pl.ds(i * L, L)] = ones
        return carry
    lax.fori_loop(0, K // L, ob, 0)
    pltpu.sync_copy(zb_v, acc_sh.at[pl.ds(s * (NDEG // NS), NDEG // NS)])
    plsc.subcore_barrier()
    def body(j, carry):
        pltpu.sync_copy(ones_v, acc_sh.at[row_v.at[j]], add=True)
        return carry
    lax.fori_loop(0, NCHUNK, body, 0)
    plsc.subcore_barrier()
    pltpu.sync_copy(acc_sh.at[pl.ds(s * (NDEG // NS), NDEG // NS)], zb_v)
    pltpu.sync_copy(zb_v,
                    out_hbm.at[pl.ds(c * NDEG + s * (NDEG // NS), NDEG // NS)])


NPASS = 2
PROWS = N // NPASS
CDUMP = PROWS * T
CFH = CDUMP + 1600
CTILE = CFH // NS
CCH = 2504
SCAP = 15


@functools.partial(
    pl.kernel,
    out_type=jax.ShapeDtypeStruct((NC * NPASS * CFH,), _f32),
    mesh=_mesh,
    scratch_types=[
        pltpu.VMEM((SEGC, K), jnp.int32),
        pltpu.VMEM((SEGC, K), jnp.int32),
        pltpu.VMEM((SEGC, K), jnp.int32),
        pltpu.VMEM((SEGC, K), _f32),
        pltpu.VMEM((2512,), _f32),
        pltpu.VMEM((2512,), _f32),
        pltpu.VMEM_SHARED((CFH,), _f32),
        pltpu.SemaphoreType.DMA,
        pltpu.SemaphoreType.DMA,
    ],
)
def _sc_buildc(fidx_hbm, col_hbm, dinv_hbm, out_hbm,
               col_sv, fidx_sv, idxp_sv, val_sv, zb_v, cb_v, acc_sh,
               sem, sem2):
    c = lax.axis_index("c")
    s = lax.axis_index("s")
    zeros = jnp.zeros((L,), _f32)
    def zf(i, carry):
        zb_v[pl.ds(i * L, L)] = zeros
        return carry
    lax.fori_loop(0, 2512 // L, zf, 0)
    dinv = dinv_hbm.at[pl.ds(c * N, N)]
    G = K // L
    for p in range(NPASS):
        def zc(i, carry):
            pltpu.sync_copy(zb_v.at[pl.ds(0, CCH)],
                            acc_sh.at[pl.ds(s * CTILE + i * CCH, CCH)])
            return carry
        lax.fori_loop(0, CTILE // CCH, zc, 0)
        plsc.subcore_barrier()
        lo = p * CDUMP
        def seg(g, carry):
            pltpu.sync_copy(col_hbm.at[c, s, pl.ds(g * SEGC, SEGC)], col_sv)
            pltpu.sync_copy(fidx_hbm.at[c, s, pl.ds(g * SEGC, SEGC)], fidx_sv)
            def gat(j, carry2):
                pltpu.async_copy(dinv.at[col_sv.at[j]], val_sv.at[j], sem)
                return carry2
            lax.fori_loop(0, SEGC, gat, 0)
            def clamp(i, carry2):
                j = i // G
                o = (i % G) * L
                fi = fidx_sv[j, pl.ds(o, L)]
                inr = (fi >= lo) & (fi < lo + CDUMP)
                dump = CDUMP + lax.rem(fi, 1536)
                idxp_sv[j, pl.ds(o, L)] = jnp.where(inr, fi - lo, dump)
                return carry2
            lax.fori_loop(0, SEGC * G, clamp, 0)
            def gatw(j, carry2):
                pltpu.make_async_copy(dinv.at[col_sv.at[j]], val_sv.at[j],
                                      sem).wait()
                return carry2
            lax.fori_loop(0, SEGC, gatw, 0)
            def scat(j, carry2):
                pltpu.async_copy(val_sv.at[j], acc_sh.at[idxp_sv.at[j]], sem2,
                                 add=True)
                return carry2
            lax.fori_loop(0, SEGC, scat, 0)
            def scatw(j, carry2):
                pltpu.make_async_copy(val_sv.at[j], acc_sh.at[idxp_sv.at[j]],
                                      sem2).wait()
                return carry2
            lax.fori_loop(0, SEGC, scatw, 0)
            return carry
        lax.fori_loop(0, NSEG, seg, 0)
        plsc.subcore_barrier()
        obase = c * (NPASS * CFH) + p * CFH + s * CTILE
        def co(i, carry):
            bounce = cb_v.at[pl.ds(0, CCH)]
            pltpu.sync_copy(acc_sh.at[pl.ds(s * CTILE + i * CCH, CCH)], bounce)
            pltpu.sync_copy(bounce, out_hbm.at[pl.ds(obase + i * CCH, CCH)])
            return carry
        lax.fori_loop(0, CTILE // CCH, co, 0)
        plsc.subcore_barrier()


@functools.partial(
    pl.kernel,
    out_type=jax.ShapeDtypeStruct((NC, SROWS, D), _f32),
    mesh=_mesh,
    scratch_types=[
        pltpu.VMEM((NCHUNK, K), jnp.int32),
        pltpu.VMEM((NCHUNK, K), jnp.int32),
        pltpu.VMEM((K, D), _f32),
        pltpu.VMEM_SHARED((SROWS, D), _f32),
        pltpu.SemaphoreType.DMA,
    ],
)
def _sc_spmm(zeros_hbm, xs_hbm, col_hbm, row_hbm, out_hbm,
             col_v, row_v, gb_v, acc_sh, sem):
    c = lax.axis_index("c")
    s = lax.axis_index("s")
    pltpu.sync_copy(col_hbm.at[c, s], col_v)
    pltpu.sync_copy(row_hbm.at[c, s], row_v)
    RL = D // L
    RPT = SROWS // NS
    zeros = jnp.zeros((L,), _f32)
    def zb(i, carry):
        gb_v[i // RL, pl.ds((i % RL) * L, L)] = zeros
        return carry
    lax.fori_loop(0, K * RL, zb, 0)
    def zc(i, carry):
        pltpu.sync_copy(gb_v, acc_sh.at[pl.ds(s * RPT + i * K, K)])
        return carry
    lax.fori_loop(0, RPT // K, zc, 0)
    plsc.subcore_barrier()
    xs = xs_hbm.at[c]
    def body(j, carry):
        pltpu.async_copy(xs.at[col_v.at[j]], gb_v, sem).wait()
        pltpu.sync_copy(gb_v, acc_sh.at[row_v.at[j]], add=True)
        return carry
    lax.fori_loop(0, NCHUNK, body, 0)
    plsc.subcore_barrier()
    def co(i, carry):
        pltpu.sync_copy(acc_sh.at[pl.ds(s * RPT + i * K, K)], gb_v)
        pltpu.sync_copy(gb_v, out_hbm.at[c, pl.ds(s * RPT + i * K, K)])
        return carry
    lax.fori_loop(0, RPT // K, co, 0)


@functools.partial(
    pl.kernel,
    out_type=(jax.ShapeDtypeStruct((BATCH, D), _f32),
              jax.ShapeDtypeStruct((BATCH, D), _f32)),
    mesh=_mesh,
    scratch_types=[
        pltpu.VMEM((BPT,), jnp.int32),
        pltpu.VMEM((BPT, D), _f32),
        pltpu.SemaphoreType.DMA,
    ],
)
def _sc_gather(x_hbm, r_hbm, sub_hbm, rel_hbm, sube_hbm, rele_hbm,
               idx_v, buf_v, sem):
    c = lax.axis_index("c")
    s = lax.axis_index("s")
    base = (s * NC + c) * BPT
    pltpu.sync_copy(sub_hbm.at[pl.ds(base, BPT)], idx_v)
    pltpu.async_copy(x_hbm.at[idx_v], buf_v, sem).wait()
    pltpu.sync_copy(buf_v, sube_hbm.at[pl.ds(base, BPT)])
    pltpu.sync_copy(rel_hbm.at[pl.ds(base, BPT)], idx_v)
    pltpu.async_copy(r_hbm.at[idx_v], buf_v, sem).wait()
    pltpu.sync_copy(buf_v, rele_hbm.at[pl.ds(base, BPT)])


def _tc_pre_body(degp_ref, x_ref, dinv_ref, xs_ref):
    deg = degp_ref[:, :N]
    dinv = jnp.where(deg > 0, lax.rsqrt(jnp.maximum(deg, 1e-12)), 0.0)
    dinv_ref[...] = dinv
    xs_ref[...] = dinv[:, :, None] * x_ref[...][None, :, :]


def _tc_pre(degp, x):
    return pl.pallas_call(
        _tc_pre_body,
        out_shape=(jax.ShapeDtypeStruct((NC, N), _f32),
                   jax.ShapeDtypeStruct((NC, N, D), _f32)),
    )(degp, x)


def _tc_scale_body(dinv_ref, x_ref, xs_ref):
    xs_ref[...] = dinv_ref[...][:, :, None] * x_ref[...][None, :, :]


def _tc_scale(dinv, x):
    return pl.pallas_call(
        _tc_scale_body,
        out_shape=jax.ShapeDtypeStruct((NC, N, D), _f32),
    )(dinv, x)


RB = 2000


def _tc_aggmm_body(x_ref, S_ref, C_ref, dinv_ref, relf_ref,
                   wl_ref, wi_ref, wo_ref, b_ref, pre_ref):
    x = x_ref[...]
    relf = relf_ref[...]
    rel200 = relf[:T, :]
    dinv = dinv_ref[0]
    res = jnp.zeros((RB, D), _f32)
    for h, w_ref in ((0, wi_ref), (1, wo_ref)):
        Rh = jnp.dot(C_ref[h], rel200, preferred_element_type=_f32)
        agg = dinv[h][:, None] * (S_ref[h] - Rh)
        res = res + jnp.dot(agg, w_ref[...], preferred_element_type=_f32)
    loop_res = jnp.dot(x - relf[T, :][None, :], wl_ref[...],
                       preferred_element_type=_f32)
    pre_ref[...] = (res + loop_res) * (1.0 / 3.0) + b_ref[...][None, :]


def _tc_bn_body(pre_ref, relf_ref, wr_ref, g_ref, be_ref, out_ref, nrel_ref):
    out = pre_ref[...]
    mean = jnp.mean(out, axis=0)
    var = jnp.mean((out - mean[None, :]) ** 2, axis=0)
    out = (out - mean[None, :]) * lax.rsqrt(var + 1e-5)[None, :] * \
        g_ref[...][None, :] + be_ref[...][None, :]
    out_ref[...] = jnp.tanh(out)
    nrel_ref[...] = jnp.dot(relf_ref[...], wr_ref[...],
                            preferred_element_type=_f32)[:T, :]


def _tc_layer(x, S, C, dinv, relf, wl, wi, wo, wr, b, g, be, want_xsn):
    del want_xsn
    full = lambda *shape: pl.BlockSpec(shape, lambda i: (0,) * len(shape))
    pre = pl.pallas_call(
        _tc_aggmm_body,
        grid=(N // RB,),
        in_specs=[
            pl.BlockSpec((RB, D), lambda i: (i, 0)),
            pl.BlockSpec((NC, RB, D), lambda i: (0, i, 0)),
            pl.BlockSpec((NC, RB, T), lambda i: (0, i, 0)),
            pl.BlockSpec((1, NC, RB), lambda i: (i, 0, 0)),
            full(T + 1, D),
            full(D, D),
            full(D, D),
            full(D, D),
            full(D),
        ],
        out_specs=pl.BlockSpec((RB, D), lambda i: (i, 0)),
        out_shape=jax.ShapeDtypeStruct((N, D), _f32),
    )(x, S, C, dinv.reshape(NC, N // RB, RB).transpose(1, 0, 2),
      relf, wl, wi, wo, b)
    return pl.pallas_call(
        _tc_bn_body,
        out_shape=(jax.ShapeDtypeStruct((N, D), _f32),
                   jax.ShapeDtypeStruct((T, D), _f32)),
    )(pre, relf, wr, g, be)


def kernel(sub, rel, edge_index, edge_type, init_embed, init_rel,
           w_loop1, w_in1, w_out1, w_rel1, loop_rel1, bias1, gamma1, beta1,
           w_loop2, w_in2, w_out2, w_rel2, loop_rel2, bias2, gamma2, beta2):
    ei = edge_index.astype(jnp.int32)
    ety = edge_type.astype(jnp.int32)
    pad_row = N + (jnp.arange(PADN, dtype=jnp.int32) % (SROWS - N))
    pad_zero = jnp.zeros((PADN,), jnp.int32)
    row_p4 = jnp.stack([jnp.concatenate([ei[0, :EH], pad_row]),
                        jnp.concatenate([ei[0, EH:], pad_row])]
                       ).reshape(NC, NS, NCHUNK, K)
    col_p4 = jnp.stack([jnp.concatenate([ei[1, :EH], pad_zero]),
                        jnp.concatenate([ei[1, EH:], pad_zero])]
                       ).reshape(NC, NS, NCHUNK, K)
    et_p4 = jnp.stack([jnp.concatenate([ety[:EH], pad_zero]),
                       jnp.concatenate([ety[EH:], pad_zero])]
                      ).reshape(NC, NS, NCHUNK, K)
    fidx_p4 = row_p4 * T + et_p4

    degp = _sc_degree(row_p4).reshape(NC, NDEG)
    dinv, xs1 = _tc_pre(degp, init_embed)
    Craw = _sc_buildc(fidx_p4, col_p4, dinv.reshape(-1))
    C = Craw.reshape(NC, NPASS, CFH)[:, :, :CDUMP].reshape(NC, N, T)

    zrows = jnp.zeros((SROWS // NS, D), _f32)
    S1 = _sc_spmm(zrows, xs1, col_p4, row_p4)[:, :N, :]
    relf1 = jnp.concatenate([init_rel, loop_rel1], axis=0)
    x1, r1 = _tc_layer(init_embed, S1, C, dinv, relf1,
                       w_loop1, w_in1, w_out1, w_rel1,
                       bias1, gamma1, beta1, False)
    xs2 = _tc_scale(dinv, x1)

    S2 = _sc_spmm(zrows, xs2, col_p4, row_p4)[:, :N, :]
    relf2 = jnp.concatenate([r1, loop_rel2], axis=0)
    x2, r2 = _tc_layer(x1, S2, C, dinv, relf2,
                       w_loop2, w_in2, w_out2, w_rel2,
                       bias2, gamma2, beta2, False)

    sub_emb, rel_emb = _sc_gather(x2, r2, sub.astype(jnp.int32),
                                  rel.astype(jnp.int32))
    return sub_emb, rel_emb, x2
